# baseline (speedup 1.0000x reference)
import dataclasses
import functools

import jax
import jax.numpy as jnp
from jax import lax
from jax.experimental import pallas as pl
from jax.experimental.pallas import tpu as pltpu
from jax.experimental.pallas import tpu_sc as plsc

_NUM_EMB = 1027
_PAD_IDX = _NUM_EMB - 1
_EMB = 128
_L = 512
_PAD_TOKEN = 3


def _tc_prep_kernel(x_ref, table_ref, starts_ref, text_ref):
    npad = jnp.sum((x_ref[...] == _PAD_TOKEN).astype(jnp.int32), axis=1)
    starts_ref[...] = _L - npad
    pad_row = table_ref[_PAD_IDX, :]
    text_ref[0:_L, :] = jnp.broadcast_to(pad_row[None, :], (_L, _EMB))
    text_ref[_L : 2 * _L, :] = table_ref[0:_L, :]


_ROW = _L * _EMB
_NC, _NS = 2, 16
_LANES = 16
_K = 128
_TT = (2 * _L - _K) * _EMB
_A = 26


def _sc_copy(text, starts, B):
    mesh = plsc.VectorSubcoreMesh(core_axis_name="c", subcore_axis_name="s")
    rpw = B // (_NC * _NS)

    cp = pltpu.CompilerParams()
    if "needs_layout_passes" in pltpu.CompilerParams.__dataclass_fields__:
        cp = dataclasses.replace(cp, needs_layout_passes=False)

    @functools.partial(
        pl.kernel,
        out_type=jax.ShapeDtypeStruct((B * _ROW,), text.dtype),
        mesh=mesh,
        compiler_params=cp,
        scratch_types=[
            pltpu.VMEM_SHARED((2 * _L * _EMB,), jnp.float32),
            pltpu.VMEM((_TT,), jnp.float32),
            pltpu.VMEM((rpw,), jnp.int32),
            pltpu.SemaphoreType.DMA,
            pltpu.SemaphoreType.DMA,
        ],
    )
    def run(text_hbm, starts_hbm, out_hbm, text_sp, text_tile_flat, s_vmem,
            sem_in, sem):
        cid = lax.axis_index("c")
        sid = lax.axis_index("s")
        wid = cid * _NS + sid
        base = wid * rpw

        @pl.when(sid == 0)
        def _stage():
            pltpu.async_copy(text_hbm, text_sp, sem_in)

        pltpu.async_copy(
            text_hbm.at[pl.ds(_K * _EMB, _TT)], text_tile_flat, sem_in
        )

        pltpu.async_copy(starts_hbm.at[pl.ds(base, rpw)], s_vmem, sem).wait()

        @pl.when(sid == 0)
        def _wait_stage():
            pltpu.make_async_copy(text_hbm, text_sp, sem_in).wait()

        pltpu.make_async_copy(
            text_hbm.at[pl.ds(_K * _EMB, _TT)], text_tile_flat, sem_in
        ).wait()

        plsc.subcore_barrier()

        lane_iota = lax.broadcasted_iota(jnp.int32, (16,), 0)

        @pl.loop(0, rpw)
        def _fire(i):
            svec = s_vmem[pl.ds((i // 16) * 16, 16)]
            s = jnp.sum(jnp.where(lane_iota == i % 16, svec, 0))
            dst = out_hbm.at[pl.ds((base + i) * _ROW, _ROW)]

            @pl.when(jnp.logical_and(i >= _A, s >= _K))
            def _from_tile():
                pltpu.async_copy(
                    text_tile_flat.at[pl.ds((s - _K) * _EMB, _ROW)], dst, sem
                )

            @pl.when(jnp.logical_or(i < _A, s < _K))
            def _from_spmem():
                pltpu.async_copy(
                    text_sp.at[pl.ds(s * _EMB, _ROW)], dst, sem
                )

        @pl.loop(0, rpw)
        def _drain(i):
            pltpu.make_async_copy(
                text_hbm.at[pl.ds(0, _ROW)],
                out_hbm.at[pl.ds((base + i) * _ROW, _ROW)],
                sem,
            ).wait()

    flat = run(text.reshape(-1), starts)
    return flat.reshape(B, _L, _EMB)


def kernel(x, table):
    B, L = x.shape
    starts, text = pl.pallas_call(
        _tc_prep_kernel,
        grid=(1,),
        in_specs=[
            pl.BlockSpec((B, L), lambda i: (0, 0)),
            pl.BlockSpec((_NUM_EMB, _EMB), lambda i: (0, 0)),
        ],
        out_specs=[
            pl.BlockSpec((B,), lambda i: (0,)),
            pl.BlockSpec((2 * _L, _EMB), lambda i: (0, 0)),
        ],
        out_shape=[
            jax.ShapeDtypeStruct((B,), jnp.int32),
            jax.ShapeDtypeStruct((2 * _L, _EMB), table.dtype),
        ],
    )(x, table)
    return _sc_copy(text, starts, B)


# speedup vs baseline: 1.3354x; 1.3354x over previous
import dataclasses
import functools

import jax
import jax.numpy as jnp
from jax import lax
from jax.experimental import pallas as pl
from jax.experimental.pallas import tpu as pltpu
from jax.experimental.pallas import tpu_sc as plsc

_NUM_EMB = 1027
_PAD_IDX = _NUM_EMB - 1
_EMB = 128
_L = 512
_PAD_TOKEN = 3


def _tc_prep_kernel(x_ref, table_ref, starts_ref, text_ref):
    npad = jnp.sum((x_ref[...] == _PAD_TOKEN).astype(jnp.int32), axis=1)
    starts_ref[...] = _L - npad
    pad_row = table_ref[_PAD_IDX, :]
    text_ref[0:_L, :] = jnp.broadcast_to(pad_row[None, :], (_L, _EMB))
    text_ref[_L : 2 * _L, :] = table_ref[0:_L, :]


_ROW = _L * _EMB
_NC, _NS = 2, 16
_LANES = 16
_K = 128
_TT = (2 * _L - _K) * _EMB
_A = 22


def _sc_copy(text, starts, B):
    mesh = plsc.VectorSubcoreMesh(core_axis_name="c", subcore_axis_name="s")
    rpw = B // (_NC * _NS)

    cp = pltpu.CompilerParams()
    if "needs_layout_passes" in pltpu.CompilerParams.__dataclass_fields__:
        cp = dataclasses.replace(cp, needs_layout_passes=False)

    @functools.partial(
        pl.kernel,
        out_type=jax.ShapeDtypeStruct((B * _ROW,), text.dtype),
        mesh=mesh,
        compiler_params=cp,
        scratch_types=[
            pltpu.VMEM_SHARED((2 * _L * _EMB,), jnp.float32),
            pltpu.VMEM((_TT,), jnp.float32),
            pltpu.VMEM((rpw,), jnp.int32),
            pltpu.SemaphoreType.DMA,
            pltpu.SemaphoreType.DMA,
        ],
    )
    def run(text_hbm, starts_hbm, out_hbm, text_sp, text_tile_flat, s_vmem,
            sem_in, sem):
        cid = lax.axis_index("c")
        sid = lax.axis_index("s")
        wid = cid * _NS + sid
        base = wid * rpw

        @pl.when(sid == 0)
        def _stage():
            pltpu.async_copy(text_hbm, text_sp, sem_in)

        pltpu.async_copy(
            text_hbm.at[pl.ds(_K * _EMB, _TT)], text_tile_flat, sem_in
        )

        pltpu.async_copy(starts_hbm.at[pl.ds(base, rpw)], s_vmem, sem).wait()

        @pl.when(sid == 0)
        def _wait_stage():
            pltpu.make_async_copy(text_hbm, text_sp, sem_in).wait()

        pltpu.make_async_copy(
            text_hbm.at[pl.ds(_K * _EMB, _TT)], text_tile_flat, sem_in
        ).wait()

        plsc.subcore_barrier()

        lane_iota = lax.broadcasted_iota(jnp.int32, (16,), 0)

        @pl.loop(0, rpw)
        def _fire(i):
            svec = s_vmem[pl.ds((i // 16) * 16, 16)]
            s = jnp.sum(jnp.where(lane_iota == i % 16, svec, 0))
            dst = out_hbm.at[pl.ds((base + i) * _ROW, _ROW)]

            @pl.when(jnp.logical_and(i >= _A, s >= _K))
            def _from_tile():
                pltpu.async_copy(
                    text_tile_flat.at[pl.ds((s - _K) * _EMB, _ROW)], dst, sem
                )

            @pl.when(jnp.logical_or(i < _A, s < _K))
            def _from_spmem():
                pltpu.async_copy(
                    text_sp.at[pl.ds(s * _EMB, _ROW)], dst, sem
                )

        @pl.loop(0, rpw)
        def _drain(i):
            pltpu.make_async_copy(
                text_hbm.at[pl.ds(0, _ROW)],
                out_hbm.at[pl.ds((base + i) * _ROW, _ROW)],
                sem,
            ).wait()

    flat = run(text.reshape(-1), starts)
    return flat.reshape(B, _L, _EMB)


def kernel(x, table):
    B, L = x.shape
    starts, text = pl.pallas_call(
        _tc_prep_kernel,
        grid=(1,),
        in_specs=[
            pl.BlockSpec((B, L), lambda i: (0, 0)),
            pl.BlockSpec((_NUM_EMB, _EMB), lambda i: (0, 0)),
        ],
        out_specs=[
            pl.BlockSpec((B,), lambda i: (0,)),
            pl.BlockSpec((2 * _L, _EMB), lambda i: (0, 0)),
        ],
        out_shape=[
            jax.ShapeDtypeStruct((B,), jnp.int32),
            jax.ShapeDtypeStruct((2 * _L, _EMB), table.dtype),
        ],
    )(x, table)
    return _sc_copy(text, starts, B)
